# Initial kernel scaffold; baseline (speedup 1.0000x reference)
import jax
import jax.numpy as jnp
from jax.experimental import pallas as pl


def kernel(x, tables):
    raise NotImplementedError("write your pallas kernel here")


# speedup vs baseline: 1.1990x; 1.1990x over previous
import functools

import jax
import jax.numpy as jnp
from jax import lax
from jax.experimental import pallas as pl
from jax.experimental.pallas import tpu as pltpu
from jax.experimental.pallas import tpu_sc as plsc

_BATCH = 16384
_INPUT_DIM = 39
_N_CONT = 13
_N_CAT = 26
_VOCAB = 100000
_EMB = 16
_OUT_DIM = _N_CONT + _N_CAT * _EMB

_NC = 2
_NS = 16
_NW = _NC * _NS
_ROWS_PER_W = _BATCH // _NW
_CHUNK = 128
_N_CHUNKS = _ROWS_PER_W // _CHUNK


def _emb_kernel(x_hbm, tab_hbm, out_hbm, x_v, idx_v, emb_v, out_v, gsem, wsem):
    wid = lax.axis_index("s") * _NC + lax.axis_index("c")
    lane = lax.broadcasted_iota(jnp.int32, (16,), 0)
    off_a = lane * _VOCAB
    off_b = (lane + 10) * _VOCAB

    for t in range(_N_CHUNKS):
        base = wid * _ROWS_PER_W + t * _CHUNK

        pltpu.sync_copy(x_hbm.at[pl.ds(base, _CHUNK), :], x_v)

        def row_body(i, _):
            col_i = jnp.full((16,), i, jnp.int32)
            xa = x_v[i, pl.ds(_N_CONT, 16)]
            xb = x_v[i, pl.ds(_N_CONT + 10, 16)]
            plsc.store_scatter(idx_v, [lane, col_i], xa + off_a)
            plsc.store_scatter(idx_v, [lane + 10, col_i], xb + off_b)
            xc = x_v[i, pl.ds(0, 16)]
            out_v[i, pl.ds(0, 16)] = xc.astype(jnp.float32)
            return 0

        lax.fori_loop(0, _CHUNK, row_body, 0)

        gathers = []
        for c in range(_N_CAT):
            d = pltpu.make_async_copy(tab_hbm.at[idx_v.at[c]], emb_v.at[c], gsem)
            d.start()
            gathers.append(d)
        for d in gathers:
            d.wait()

        def asm_body(i, _):
            for c in range(_N_CAT):
                out_v[i, pl.ds(_N_CONT + _EMB * c, _EMB)] = emb_v[c, i, :]
            return 0

        lax.fori_loop(0, _CHUNK, asm_body, 0)

        pltpu.sync_copy(out_v, out_hbm.at[pl.ds(base, _CHUNK), :])


@jax.jit
def _run(x, tab_flat):
    mesh = plsc.VectorSubcoreMesh(core_axis_name="c", subcore_axis_name="s")
    f = functools.partial(
        pl.kernel,
        mesh=mesh,
        out_type=jax.ShapeDtypeStruct((_BATCH, _OUT_DIM), jnp.float32),
        scratch_types=[
            pltpu.VMEM((_CHUNK, _INPUT_DIM), jnp.int32),
            pltpu.VMEM((_N_CAT, _CHUNK), jnp.int32),
            pltpu.VMEM((_N_CAT, _CHUNK, _EMB), jnp.float32),
            pltpu.VMEM((_CHUNK, _OUT_DIM), jnp.float32),
            pltpu.SemaphoreType.DMA,
            pltpu.SemaphoreType.DMA,
        ],
        compiler_params=pltpu.CompilerParams(
            use_tc_tiling_on_sc=False, needs_layout_passes=False
        ),
    )(_emb_kernel)
    return f(x, tab_flat)


def kernel(x, tables):
    tab_flat = tables.reshape(_N_CAT * _VOCAB, _EMB)
    return _run(x, tab_flat)


# speedup vs baseline: 1.2020x; 1.0025x over previous
import functools

import jax
import jax.numpy as jnp
from jax import lax
from jax.experimental import pallas as pl
from jax.experimental.pallas import tpu as pltpu
from jax.experimental.pallas import tpu_sc as plsc

_BATCH = 16384
_INPUT_DIM = 39
_N_CONT = 13
_N_CAT = 26
_VOCAB = 100000
_EMB = 16
_OUT_DIM = _N_CONT + _N_CAT * _EMB

_NC = 2
_NS = 16
_NW = _NC * _NS
_ROWS_PER_W = _BATCH // _NW
_CHUNK = 128
_N_CHUNKS = _ROWS_PER_W // _CHUNK


def _emb_kernel(x_hbm, tab_hbm, out_hbm, x_v, idx_v, emb_v, out_v, gsem, wsem):
    wid = lax.axis_index("s") * _NC + lax.axis_index("c")
    lane = lax.broadcasted_iota(jnp.int32, (16,), 0)

    for t in range(_N_CHUNKS):
        base = wid * _ROWS_PER_W + t * _CHUNK

        pltpu.sync_copy(x_hbm.at[pl.ds(base, _CHUNK), :], x_v)

        def row_body(i, _):
            col_i = jnp.full((16,), i, jnp.int32)
            xa = x_v[i, pl.ds(_N_CONT, 16)]
            xb = x_v[i, pl.ds(_N_CONT + 10, 16)]
            plsc.store_scatter(idx_v, [lane, col_i], xa)
            plsc.store_scatter(idx_v, [lane + 10, col_i], xb)
            xc = x_v[i, pl.ds(0, 16)]
            out_v[i, pl.ds(0, 16)] = xc.astype(jnp.float32)
            return 0

        lax.fori_loop(0, _CHUNK, row_body, 0)

        gathers = []
        for c in range(_N_CAT):
            d = pltpu.make_async_copy(
                tab_hbm.at[c].at[idx_v.at[c]], emb_v.at[c], gsem
            )
            d.start()
            gathers.append(d)
        for d in gathers:
            d.wait()

        def asm_body(i, _):
            for c in range(_N_CAT):
                out_v[i, pl.ds(_N_CONT + _EMB * c, _EMB)] = emb_v[c, i, :]
            return 0

        lax.fori_loop(0, _CHUNK, asm_body, 0)

        pltpu.sync_copy(out_v, out_hbm.at[pl.ds(base, _CHUNK), :])


@jax.jit
def _run(x, tables):
    mesh = plsc.VectorSubcoreMesh(core_axis_name="c", subcore_axis_name="s")
    f = functools.partial(
        pl.kernel,
        mesh=mesh,
        out_type=jax.ShapeDtypeStruct((_BATCH, _OUT_DIM), jnp.float32),
        scratch_types=[
            pltpu.VMEM((_CHUNK, _INPUT_DIM), jnp.int32),
            pltpu.VMEM((_N_CAT, _CHUNK), jnp.int32),
            pltpu.VMEM((_N_CAT, _CHUNK, _EMB), jnp.float32),
            pltpu.VMEM((_CHUNK, _OUT_DIM), jnp.float32),
            pltpu.SemaphoreType.DMA,
            pltpu.SemaphoreType.DMA,
        ],
        compiler_params=pltpu.CompilerParams(
            use_tc_tiling_on_sc=False, needs_layout_passes=False
        ),
    )(_emb_kernel)
    return f(x, tables)


def kernel(x, tables):
    return _run(x, tables)
